# baseline (speedup 1.0000x reference)
import functools

import jax
import jax.numpy as jnp
from jax.experimental import pallas as pl
from jax.experimental.pallas import tpu as pltpu
from jax.experimental.pallas import tpu_sc as plsc

COMMITMENT_COST = 0.25

N = 8192
C = 8192
D = 256

TN = 512
NI = N // TN


def _vq_body(x_ref, emb_ref, x2_ref, e2_ref, dist_ref, idx_ref, lossp_ref):
    x = x_ref[...]
    emb = emb_ref[...]

    xy = jax.lax.dot_general(
        x, emb, (((1,), (1,)), ((), ())),
        preferred_element_type=jnp.float32)
    x2 = x2_ref[...]
    e2 = e2_ref[0]
    d2 = jnp.maximum(x2 + e2 - 2.0 * xy, 0.0)
    vals = -jnp.sqrt(d2)
    dist_ref[...] = vals

    bmax = jnp.max(vals, axis=1, keepdims=True)
    col = jax.lax.broadcasted_iota(jnp.int32, (TN, C), 1)
    barg = jnp.min(jnp.where(vals == bmax, col, jnp.int32(C)),
                   axis=1, keepdims=True)
    idx_ref[...] = barg.reshape(TN)
    lossp_ref[0, 0, 0] = jnp.sum(bmax * bmax)


def _vq_distances(xt, emb, x2, e2):
    return pl.pallas_call(
        _vq_body,
        grid=(NI,),
        in_specs=[
            pl.BlockSpec((TN, D), lambda i: (i, 0)),
            pl.BlockSpec((C, D), lambda i: (0, 0)),
            pl.BlockSpec((TN, 1), lambda i: (i, 0)),
            pl.BlockSpec((1, 1, C), lambda i: (0, 0, 0)),
        ],
        out_specs=[
            pl.BlockSpec((TN, C), lambda i: (i, 0)),
            pl.BlockSpec((TN,), lambda i: (i,)),
            pl.BlockSpec((1, 1, 1), lambda i: (i, 0, 0), memory_space=pltpu.SMEM),
        ],
        out_shape=[
            jax.ShapeDtypeStruct((N, C), jnp.float32),
            jax.ShapeDtypeStruct((N,), jnp.int32),
            jax.ShapeDtypeStruct((NI, 1, 1), jnp.float32),
        ],
    )(xt, emb, x2, e2)


def _sc_gather(emb, idx):
    info = plsc.get_sparse_core_info()
    nworkers = info.num_cores * info.num_subcores
    bpw = N // nworkers
    mesh = plsc.VectorSubcoreMesh(core_axis_name="c", subcore_axis_name="s")

    @functools.partial(
        pl.kernel, mesh=mesh,
        out_type=jax.ShapeDtypeStruct((N, D), jnp.float32),
        scratch_types=[
            pltpu.VMEM((bpw,), jnp.int32),
            pltpu.VMEM((bpw, D), jnp.float32),
            pltpu.SemaphoreType.DMA,
        ],
    )
    def k(emb_hbm, idx_hbm, out_hbm, idx_v, rows_v, sem):
        wid = jax.lax.axis_index("s") * info.num_cores + jax.lax.axis_index("c")
        base = wid * bpw
        pltpu.sync_copy(idx_hbm.at[pl.ds(base, bpw)], idx_v)
        pltpu.async_copy(emb_hbm.at[idx_v], rows_v, sem).wait()
        pltpu.sync_copy(rows_v, out_hbm.at[pl.ds(base, bpw)])

    return k(emb, idx)


def kernel(inputs, embedding):
    x = jnp.transpose(inputs, (0, 2, 1))
    emb = embedding[0]
    x2 = jnp.sum(x * x, axis=-1, keepdims=True)[0]
    e2 = jnp.sum(embedding * embedding, axis=-1)[:, None, :]

    dist, idx_flat, lossp = _vq_distances(x[0], emb, x2, e2)
    quant = _sc_gather(emb, idx_flat)

    out = jnp.transpose(quant)[None]
    encoding_indices = idx_flat[None]
    loss = (jnp.sum(lossp) * (COMMITMENT_COST / (N * D))).reshape(())
    distances = dist[None]
    return (out, encoding_indices, loss, distances)
